# baseline (speedup 1.0000x reference)
import functools

import jax
import jax.numpy as jnp
from jax import lax
from jax.experimental import pallas as pl
from jax.experimental.pallas import tpu as pltpu
from jax.experimental.pallas import tpu_sc as plsc

N = 10000
E = 320000
G = 128
EPS = 1e-5

NC = 2
NS = 16
EB = 128
E_PAD = 327680
E_PER_TILE = E_PAD // (NC * NS)
N_CHUNKS = E_PER_TILE // EB
NP = 10240
ROWS_PER_TILE = NP // NS
ZR = 16
IG = 5
GC = N_CHUNKS // IG

_MESH = plsc.VectorSubcoreMesh(core_axis_name="c", subcore_axis_name="s")


def _zero_fill(buf, rows, d):
    zero16 = jnp.zeros((16,), jnp.float32)

    @pl.loop(0, rows)
    def _(r):
        @pl.loop(0, d, step=16)
        def _(j):
            buf[r, pl.ds(j, 16)] = zero16


@functools.cache
def _sc_degree_kernel():
    @functools.partial(
        pl.kernel,
        mesh=_MESH,
        out_type=jax.ShapeDtypeStruct((NC, NP, 128), jnp.float32),
        scratch_types=[
            pltpu.VMEM_SHARED((NP, 128), jnp.float32),
            pltpu.VMEM((EB, 128), jnp.float32),
            pltpu.VMEM((GC, EB), jnp.int32),
            pltpu.VMEM((ZR, 128), jnp.float32),
            pltpu.SemaphoreType.DMA,
        ],
    )
    def k(dst_hbm, out_hbm, acc_sh, ones_v, di_v, z_v, sem):
        c = lax.axis_index("c")
        s = lax.axis_index("s")
        one16 = jnp.ones((16,), jnp.float32)

        @pl.loop(0, EB)
        def _(r):
            @pl.loop(0, 128, step=16)
            def _(j):
                ones_v[r, pl.ds(j, 16)] = one16

        _zero_fill(z_v, ZR, 128)

        @pl.loop(0, ROWS_PER_TILE, step=ZR)
        def _(r0):
            pltpu.sync_copy(z_v, acc_sh.at[pl.ds(s * ROWS_PER_TILE + r0, ZR)])

        plsc.subcore_barrier()

        @pl.loop(0, IG)
        def _(g):
            pltpu.sync_copy(dst_hbm.at[c, s, g], di_v)
            for h in range(GC // 8):
                hs_ = [pltpu.async_copy(ones_v, acc_sh.at[di_v.at[h * 8 + j]],
                                        sem, add=True) for j in range(8)]
                for cp in hs_:
                    cp.wait()

        plsc.subcore_barrier()
        pltpu.sync_copy(
            acc_sh.at[pl.ds(s * ROWS_PER_TILE, ROWS_PER_TILE)],
            out_hbm.at[c, pl.ds(s * ROWS_PER_TILE, ROWS_PER_TILE)],
        )

    return k


@functools.cache
def _sc_scatter_kernel(d):
    @functools.partial(
        pl.kernel,
        mesh=_MESH,
        out_type=jax.ShapeDtypeStruct((NC, NP, d), jnp.float32),
        scratch_types=[
            pltpu.VMEM_SHARED((NP, d), jnp.float32),
            pltpu.VMEM((EB, d), jnp.float32),
            pltpu.VMEM((EB, d), jnp.float32),
            pltpu.VMEM((GC, EB), jnp.int32),
            pltpu.VMEM((GC, EB), jnp.int32),
            pltpu.VMEM((ZR, d), jnp.float32),
            pltpu.SemaphoreType.DMA,
            pltpu.SemaphoreType.DMA,
        ],
    )
    def k(hs_hbm, src_hbm, dst_hbm, out_hbm, acc_sh, rows0, rows1, si_v, di_v,
          z_v, sem0, sem1):
        c = lax.axis_index("c")
        s = lax.axis_index("s")

        _zero_fill(z_v, ZR, d)

        @pl.loop(0, ROWS_PER_TILE, step=ZR)
        def _(r0):
            pltpu.sync_copy(z_v, acc_sh.at[pl.ds(s * ROWS_PER_TILE + r0, ZR)])

        plsc.subcore_barrier()

        @pl.loop(0, IG)
        def _(g):
            pltpu.sync_copy(src_hbm.at[c, s, g], si_v)
            pltpu.sync_copy(dst_hbm.at[c, s, g], di_v)
            pltpu.async_copy(hs_hbm.at[si_v.at[0]], rows0, sem0)

            @pl.loop(0, GC // 2)
            def _(kk):
                i0 = 2 * kk
                pltpu.make_async_copy(hs_hbm.at[si_v.at[i0]], rows0,
                                      sem0).wait()
                pltpu.async_copy(hs_hbm.at[si_v.at[i0 + 1]], rows1, sem1)
                pltpu.sync_copy(rows0, acc_sh.at[di_v.at[i0]], add=True)
                pltpu.make_async_copy(hs_hbm.at[si_v.at[i0 + 1]], rows1,
                                      sem1).wait()

                @pl.when(kk < GC // 2 - 1)
                def _():
                    pltpu.async_copy(hs_hbm.at[si_v.at[i0 + 2]], rows0, sem0)

                pltpu.sync_copy(rows1, acc_sh.at[di_v.at[i0 + 1]], add=True)

        plsc.subcore_barrier()
        pltpu.sync_copy(
            acc_sh.at[pl.ds(s * ROWS_PER_TILE, ROWS_PER_TILE)],
            out_hbm.at[c, pl.ds(s * ROWS_PER_TILE, ROWS_PER_TILE)],
        )

    return k


def _tc_matmul(x, w):
    def body(x_ref, w_ref, o_ref):
        o_ref[...] = jnp.dot(x_ref[...], w_ref[...],
                             preferred_element_type=jnp.float32)

    return pl.pallas_call(
        body,
        out_shape=jax.ShapeDtypeStruct((x.shape[0], w.shape[1]), jnp.float32),
    )(x, w)


def _tc_prescale(counts, x):
    def body(cnt_ref, x_ref, dinv_ref, xs_ref):
        deg = cnt_ref[0, :N, 0:1] + cnt_ref[1, :N, 0:1] + 1.0
        dinv = lax.rsqrt(deg)
        dinv_ref[...] = dinv
        xs_ref[...] = x_ref[...] * dinv

    return pl.pallas_call(
        body,
        out_shape=(
            jax.ShapeDtypeStruct((N, 1), jnp.float32),
            jax.ShapeDtypeStruct((N, x.shape[1]), jnp.float32),
        ),
    )(counts, x)


def _tc_layer1(acc, xs, dinv, w1, b1, g1, be1, w2):
    def body(acc_ref, xs_ref, dinv_ref, w1_ref, b_ref, g_ref, be_ref, w2_ref,
             o_ref):
        px = (acc_ref[0, :N] + acc_ref[1, :N] + xs_ref[...]) * dinv_ref[...]
        t = jnp.dot(px, w1_ref[...],
                    preferred_element_type=jnp.float32) + b_ref[...]
        mean = jnp.mean(t, axis=0, keepdims=True)
        var = jnp.mean((t - mean) ** 2, axis=0, keepdims=True)
        t = (t - mean) * lax.rsqrt(var + EPS) * g_ref[...] + be_ref[...]
        t = jnp.maximum(t, 0.0)
        o_ref[...] = jnp.dot(t, w2_ref[...],
                             preferred_element_type=jnp.float32) * dinv_ref[...]

    return pl.pallas_call(
        body,
        out_shape=jax.ShapeDtypeStruct((N, w2.shape[1]), jnp.float32),
    )(acc, xs, dinv, w1, b1.reshape(1, -1), g1.reshape(1, -1),
      be1.reshape(1, -1), w2)


def _tc_layer2(acc, hs2, dinv, b2, g2, be2):
    def body(acc_ref, hs_ref, dinv_ref, b_ref, g_ref, be_ref, o_ref):
        t = (acc_ref[0, :N] + acc_ref[1, :N] + hs_ref[...]) * dinv_ref[...] \
            + b_ref[...]
        mean = jnp.mean(t, axis=0, keepdims=True)
        var = jnp.mean((t - mean) ** 2, axis=0, keepdims=True)
        t = (t - mean) * lax.rsqrt(var + EPS) * g_ref[...] + be_ref[...]
        t = jnp.maximum(t, 0.0)
        o_ref[...] = t * dinv_ref[...]

    return pl.pallas_call(
        body,
        out_shape=jax.ShapeDtypeStruct((N, hs2.shape[1]), jnp.float32),
    )(acc, hs2, dinv, b2.reshape(1, -1), g2.reshape(1, -1), be2.reshape(1, -1))


def _tc_finish(acc, hs3, dinv, w3, b3, batch2d):
    def body(acc_ref, hs_ref, dinv_ref, w3_ref, b_ref, batch_ref, o_ref):
        ph = (acc_ref[0, :N] + acc_ref[1, :N] + hs_ref[...]) * dinv_ref[...]
        h3 = jnp.dot(ph, w3_ref[...],
                     preferred_element_type=jnp.float32) + b_ref[...]
        gids = lax.broadcasted_iota(jnp.int32, (N, G), 1)
        onehot = (batch_ref[...] == gids).astype(jnp.float32)
        sums = lax.dot_general(onehot, h3, (((0,), (0,)), ((), ())),
                               preferred_element_type=jnp.float32)
        cnts = jnp.sum(onehot, axis=0)[:, None]
        pooled = sums / jnp.maximum(cnts, 1.0)
        m = jnp.max(pooled, axis=1, keepdims=True)
        z = pooled - m
        o_ref[...] = z - jnp.log(jnp.sum(jnp.exp(z), axis=1, keepdims=True))

    return pl.pallas_call(
        body,
        out_shape=jax.ShapeDtypeStruct((G, w3.shape[1]), jnp.float32),
    )(acc, hs3, dinv, w3, b3.reshape(1, -1), batch2d)


def kernel(x, edge_index, batch, W1, b1, g1, be1, W2, b2, g2, be2, W3, b3):
    pad = E_PAD - E
    pad_src = jnp.zeros((pad,), jnp.int32)
    pad_dst = N + (jnp.arange(pad, dtype=jnp.int32) % (NP - N))
    src = jnp.concatenate([edge_index[0], pad_src]).reshape(NC, NS, IG, GC, EB)
    dst = jnp.concatenate([edge_index[1], pad_dst]).reshape(NC, NS, IG, GC, EB)
    batch2d = batch.reshape(N, 1)

    counts = _sc_degree_kernel()(dst)
    dinv, xs = _tc_prescale(counts, x)

    acc1 = _sc_scatter_kernel(128)(xs, src, dst)
    hs2 = _tc_layer1(acc1, xs, dinv, W1, b1, g1, be1, W2)

    acc2 = _sc_scatter_kernel(128)(hs2, src, dst)
    hs3 = _tc_layer2(acc2, hs2, dinv, b2, g2, be2)

    acc3 = _sc_scatter_kernel(128)(hs3, src, dst)
    return _tc_finish(acc3, hs3, dinv, W3, b3, batch2d)


# speedup vs baseline: 2.6124x; 2.6124x over previous
import functools

import jax
import jax.numpy as jnp
from jax import lax
from jax.experimental import pallas as pl
from jax.experimental.pallas import tpu as pltpu
from jax.experimental.pallas import tpu_sc as plsc

N = 10000
E = 320000
G = 128
EPS = 1e-5

NC = 2
NS = 16
EB = 128
E_PAD = 327680
E_PER_TILE = E_PAD // (NC * NS)
N_CHUNKS = E_PER_TILE // EB
NP = 10240
ROWS_PER_TILE = NP // NS
ZR = 16
IG = 5
GC = N_CHUNKS // IG

_MESH = plsc.VectorSubcoreMesh(core_axis_name="c", subcore_axis_name="s")


def _zero_fill(buf, rows, d):
    zero16 = jnp.zeros((16,), jnp.float32)

    @pl.loop(0, rows)
    def _(r):
        @pl.loop(0, d, step=16)
        def _(j):
            buf[r, pl.ds(j, 16)] = zero16


@functools.cache
def _sc_degree_kernel():
    @functools.partial(
        pl.kernel,
        mesh=_MESH,
        out_type=jax.ShapeDtypeStruct((NC, NP, 128), jnp.float32),
        scratch_types=[
            pltpu.VMEM_SHARED((NP, 128), jnp.float32),
            pltpu.VMEM((EB, 128), jnp.float32),
            pltpu.VMEM((GC, EB), jnp.int32),
            pltpu.VMEM((ZR, 128), jnp.float32),
            pltpu.SemaphoreType.DMA,
        ],
    )
    def k(dst_hbm, out_hbm, acc_sh, ones_v, di_v, z_v, sem):
        c = lax.axis_index("c")
        s = lax.axis_index("s")
        one16 = jnp.ones((16,), jnp.float32)

        @pl.loop(0, EB)
        def _(r):
            @pl.loop(0, 128, step=16)
            def _(j):
                ones_v[r, pl.ds(j, 16)] = one16

        _zero_fill(z_v, ZR, 128)

        @pl.loop(0, ROWS_PER_TILE, step=ZR)
        def _(r0):
            pltpu.sync_copy(z_v, acc_sh.at[pl.ds(s * ROWS_PER_TILE + r0, ZR)])

        plsc.subcore_barrier()

        @pl.loop(0, IG)
        def _(g):
            pltpu.sync_copy(dst_hbm.at[c, s, g], di_v)
            for h in range(GC // 8):
                hs_ = [pltpu.async_copy(ones_v, acc_sh.at[di_v.at[h * 8 + j]],
                                        sem, add=True) for j in range(8)]
                for cp in hs_:
                    cp.wait()

        plsc.subcore_barrier()
        pltpu.sync_copy(
            acc_sh.at[pl.ds(s * ROWS_PER_TILE, ROWS_PER_TILE)],
            out_hbm.at[c, pl.ds(s * ROWS_PER_TILE, ROWS_PER_TILE)],
        )

    return k


@functools.cache
def _sc_scatter_kernel(d):
    @functools.partial(
        pl.kernel,
        mesh=_MESH,
        out_type=jax.ShapeDtypeStruct((NC, NP, d), jnp.float32),
        scratch_types=[
            pltpu.VMEM_SHARED((NP, d), jnp.float32),
            pltpu.VMEM((EB, d), jnp.float32),
            pltpu.VMEM((EB, d), jnp.float32),
            pltpu.VMEM((GC, EB), jnp.int32),
            pltpu.VMEM((GC, EB), jnp.int32),
            pltpu.VMEM((ZR, d), jnp.float32),
            pltpu.SemaphoreType.DMA,
            pltpu.SemaphoreType.DMA,
        ],
    )
    def k(hs_hbm, src_hbm, dst_hbm, out_hbm, acc_sh, rows0, rows1, si_v, di_v,
          z_v, sem0, sem1):
        c = lax.axis_index("c")
        s = lax.axis_index("s")

        _zero_fill(z_v, ZR, d)

        @pl.loop(0, ROWS_PER_TILE, step=ZR)
        def _(r0):
            pltpu.sync_copy(z_v, acc_sh.at[pl.ds(s * ROWS_PER_TILE + r0, ZR)])

        plsc.subcore_barrier()

        @pl.loop(0, IG)
        def _(g):
            pltpu.sync_copy(src_hbm.at[c, s, g], si_v)
            pltpu.sync_copy(dst_hbm.at[c, s, g], di_v)
            pltpu.async_copy(hs_hbm.at[si_v.at[0]], rows0, sem0)

            @pl.loop(0, GC // 2)
            def _(kk):
                i0 = 2 * kk
                pltpu.make_async_copy(hs_hbm.at[si_v.at[i0]], rows0,
                                      sem0).wait()
                pltpu.async_copy(hs_hbm.at[si_v.at[i0 + 1]], rows1, sem1)
                pltpu.sync_copy(rows0, acc_sh.at[di_v.at[i0]], add=True)
                pltpu.make_async_copy(hs_hbm.at[si_v.at[i0 + 1]], rows1,
                                      sem1).wait()

                @pl.when(kk < GC // 2 - 1)
                def _():
                    pltpu.async_copy(hs_hbm.at[si_v.at[i0 + 2]], rows0, sem0)

                pltpu.sync_copy(rows1, acc_sh.at[di_v.at[i0 + 1]], add=True)

        plsc.subcore_barrier()
        pltpu.sync_copy(
            acc_sh.at[pl.ds(s * ROWS_PER_TILE, ROWS_PER_TILE)],
            out_hbm.at[c, pl.ds(s * ROWS_PER_TILE, ROWS_PER_TILE)],
        )

    return k


def _tc_matmul(x, w):
    def body(x_ref, w_ref, o_ref):
        o_ref[...] = jnp.dot(x_ref[...], w_ref[...],
                             preferred_element_type=jnp.float32)

    return pl.pallas_call(
        body,
        out_shape=jax.ShapeDtypeStruct((x.shape[0], w.shape[1]), jnp.float32),
    )(x, w)


def _tc_prescale(counts, x):
    def body(cnt_ref, x_ref, dinv_ref, xs_ref):
        deg = cnt_ref[0, :N, 0:1] + cnt_ref[1, :N, 0:1] + 1.0
        dinv = lax.rsqrt(deg)
        dinv_ref[...] = dinv
        xs_ref[...] = x_ref[...] * dinv

    return pl.pallas_call(
        body,
        out_shape=(
            jax.ShapeDtypeStruct((N, 1), jnp.float32),
            jax.ShapeDtypeStruct((N, x.shape[1]), jnp.float32),
        ),
    )(counts, x)


def _tc_layer1(acc, xs, dinv, w1, b1, g1, be1, w2):
    def body(acc_ref, xs_ref, dinv_ref, w1_ref, b_ref, g_ref, be_ref, w2_ref,
             o_ref):
        px = (acc_ref[0, :N] + acc_ref[1, :N] + xs_ref[...]) * dinv_ref[...]
        t = jnp.dot(px, w1_ref[...],
                    preferred_element_type=jnp.float32) + b_ref[...]
        mean = jnp.mean(t, axis=0, keepdims=True)
        var = jnp.mean((t - mean) ** 2, axis=0, keepdims=True)
        t = (t - mean) * lax.rsqrt(var + EPS) * g_ref[...] + be_ref[...]
        t = jnp.maximum(t, 0.0)
        o_ref[...] = jnp.dot(t, w2_ref[...],
                             preferred_element_type=jnp.float32) * dinv_ref[...]

    return pl.pallas_call(
        body,
        out_shape=jax.ShapeDtypeStruct((N, w2.shape[1]), jnp.float32),
    )(acc, xs, dinv, w1, b1.reshape(1, -1), g1.reshape(1, -1),
      be1.reshape(1, -1), w2)


def _tc_layer2(acc, hs2, dinv, b2, g2, be2):
    def body(acc_ref, hs_ref, dinv_ref, b_ref, g_ref, be_ref, o_ref):
        t = (acc_ref[0, :N] + acc_ref[1, :N] + hs_ref[...]) * dinv_ref[...] \
            + b_ref[...]
        mean = jnp.mean(t, axis=0, keepdims=True)
        var = jnp.mean((t - mean) ** 2, axis=0, keepdims=True)
        t = (t - mean) * lax.rsqrt(var + EPS) * g_ref[...] + be_ref[...]
        t = jnp.maximum(t, 0.0)
        o_ref[...] = t * dinv_ref[...]

    return pl.pallas_call(
        body,
        out_shape=jax.ShapeDtypeStruct((N, hs2.shape[1]), jnp.float32),
    )(acc, hs2, dinv, b2.reshape(1, -1), g2.reshape(1, -1), be2.reshape(1, -1))


def _tc_finish(acc, hs3, dinv, w3, b3, batch2d):
    def body(acc_ref, hs_ref, dinv_ref, w3_ref, b_ref, batch_ref, o_ref):
        ph = (acc_ref[0, :N] + acc_ref[1, :N] + hs_ref[...]) * dinv_ref[...]
        h3 = jnp.dot(ph, w3_ref[...],
                     preferred_element_type=jnp.float32) + b_ref[...]
        gids = lax.broadcasted_iota(jnp.int32, (N, G), 1)
        onehot = (batch_ref[...] == gids).astype(jnp.float32)
        sums = lax.dot_general(onehot, h3, (((0,), (0,)), ((), ())),
                               preferred_element_type=jnp.float32)
        cnts = jnp.sum(onehot, axis=0)[:, None]
        pooled = sums / jnp.maximum(cnts, 1.0)
        m = jnp.max(pooled, axis=1, keepdims=True)
        z = pooled - m
        o_ref[...] = z - jnp.log(jnp.sum(jnp.exp(z), axis=1, keepdims=True))

    return pl.pallas_call(
        body,
        out_shape=jax.ShapeDtypeStruct((G, w3.shape[1]), jnp.float32),
    )(acc, hs3, dinv, w3, b3.reshape(1, -1), batch2d)


def kernel(x, edge_index, batch, W1, b1, g1, be1, W2, b2, g2, be2, W3, b3):
    nt = NC * NS
    pad = E_PER_TILE - E // nt
    pr = jnp.broadcast_to(jnp.arange(pad, dtype=jnp.int32)[None], (nt, pad))
    src = jnp.concatenate([edge_index[0].reshape(nt, -1), pr], axis=1)
    dst = jnp.concatenate([edge_index[1].reshape(nt, -1), N + pr], axis=1)
    src = src.reshape(NC, NS, IG, GC, EB)
    dst = dst.reshape(NC, NS, IG, GC, EB)
    batch2d = batch.reshape(N, 1)

    counts = _sc_degree_kernel()(dst)
    dinv, xs = _tc_prescale(counts, x)

    acc1 = _sc_scatter_kernel(128)(xs, src, dst)
    hs2 = _tc_layer1(acc1, xs, dinv, W1, b1, g1, be1, W2)

    acc2 = _sc_scatter_kernel(128)(hs2, src, dst)
    hs3 = _tc_layer2(acc2, hs2, dinv, b2, g2, be2)

    acc3 = _sc_scatter_kernel(128)(hs3, src, dst)
    return _tc_finish(acc3, hs3, dinv, W3, b3, batch2d)


# speedup vs baseline: 3.1814x; 1.2178x over previous
import functools

import jax
import jax.numpy as jnp
from jax import lax
from jax.experimental import pallas as pl
from jax.experimental.pallas import tpu as pltpu
from jax.experimental.pallas import tpu_sc as plsc

N = 10000
E = 320000
G = 128
EPS = 1e-5

NC = 2
NS = 16
EB = 64
E_PAD = 327680
E_PER_TILE = E_PAD // (NC * NS)
N_CHUNKS = E_PER_TILE // EB
NP = 10240
ROWS_PER_TILE = NP // NS
ZR = 16
IG = 4
GC = N_CHUNKS // IG
NB = 4

_MESH = plsc.VectorSubcoreMesh(core_axis_name="c", subcore_axis_name="s")


def _zero_fill(buf, rows, d):
    zero16 = jnp.zeros((16,), jnp.float32)

    @pl.loop(0, rows)
    def _(r):
        @pl.loop(0, d, step=16)
        def _(j):
            buf[r, pl.ds(j, 16)] = zero16


@functools.cache
def _sc_degree_kernel():
    @functools.partial(
        pl.kernel,
        mesh=_MESH,
        out_type=jax.ShapeDtypeStruct((NC, NP, 128), jnp.float32),
        scratch_types=[
            pltpu.VMEM_SHARED((NP, 128), jnp.float32),
            pltpu.VMEM((EB, 128), jnp.float32),
            pltpu.VMEM((GC, EB), jnp.int32),
            pltpu.VMEM((ZR, 128), jnp.float32),
            pltpu.SemaphoreType.DMA,
        ],
    )
    def k(dst_hbm, out_hbm, acc_sh, ones_v, di_v, z_v, sem):
        c = lax.axis_index("c")
        s = lax.axis_index("s")
        one16 = jnp.ones((16,), jnp.float32)

        @pl.loop(0, EB)
        def _(r):
            @pl.loop(0, 128, step=16)
            def _(j):
                ones_v[r, pl.ds(j, 16)] = one16

        _zero_fill(z_v, ZR, 128)

        @pl.loop(0, ROWS_PER_TILE, step=ZR)
        def _(r0):
            pltpu.sync_copy(z_v, acc_sh.at[pl.ds(s * ROWS_PER_TILE + r0, ZR)])

        plsc.subcore_barrier()

        @pl.loop(0, IG)
        def _(g):
            pltpu.sync_copy(dst_hbm.at[c, s, g], di_v)
            for h in range(GC // 8):
                hs_ = [pltpu.async_copy(ones_v, acc_sh.at[di_v.at[h * 8 + j]],
                                        sem, add=True) for j in range(8)]
                for cp in hs_:
                    cp.wait()

        plsc.subcore_barrier()
        pltpu.sync_copy(
            acc_sh.at[pl.ds(s * ROWS_PER_TILE, ROWS_PER_TILE)],
            out_hbm.at[c, pl.ds(s * ROWS_PER_TILE, ROWS_PER_TILE)],
        )

    return k


@functools.cache
def _sc_scatter_kernel(d):
    @functools.partial(
        pl.kernel,
        mesh=_MESH,
        out_type=jax.ShapeDtypeStruct((NC, NP, d), jnp.float32),
        scratch_types=[
            pltpu.VMEM_SHARED((NP, d), jnp.float32),
        ] + [pltpu.VMEM((EB, d), jnp.float32) for _ in range(NB)] + [
            pltpu.VMEM((GC, EB), jnp.int32),
            pltpu.VMEM((GC, EB), jnp.int32),
            pltpu.VMEM((ZR, d), jnp.float32),
        ] + [pltpu.SemaphoreType.DMA for _ in range(NB)],
    )
    def k(hs_hbm, src_hbm, dst_hbm, out_hbm, acc_sh, b0, b1, b2, b3, si_v,
          di_v, z_v, s0, s1, s2, s3):
        c = lax.axis_index("c")
        s = lax.axis_index("s")
        bufs = [b0, b1, b2, b3]
        sems = [s0, s1, s2, s3]

        _zero_fill(z_v, ZR, d)

        @pl.loop(0, ROWS_PER_TILE, step=ZR)
        def _(r0):
            pltpu.sync_copy(z_v, acc_sh.at[pl.ds(s * ROWS_PER_TILE + r0, ZR)])

        plsc.subcore_barrier()

        @pl.loop(0, IG)
        def _(g):
            pltpu.sync_copy(src_hbm.at[c, s, g], si_v)
            pltpu.sync_copy(dst_hbm.at[c, s, g], di_v)
            for j in range(NB - 1):
                pltpu.async_copy(hs_hbm.at[si_v.at[j]], bufs[j], sems[j])

            @pl.loop(0, GC // NB)
            def _(kk):
                i0 = NB * kk
                for j in range(NB):
                    i = i0 + j
                    pltpu.make_async_copy(hs_hbm.at[si_v.at[i]], bufs[j],
                                          sems[j]).wait()
                    nxt = i + NB - 1
                    jn = (j + NB - 1) % NB

                    @pl.when(nxt < GC)
                    def _():
                        pltpu.async_copy(hs_hbm.at[si_v.at[nxt]], bufs[jn],
                                         sems[jn])

                    pltpu.sync_copy(bufs[j], acc_sh.at[di_v.at[i]], add=True)

        plsc.subcore_barrier()
        pltpu.sync_copy(
            acc_sh.at[pl.ds(s * ROWS_PER_TILE, ROWS_PER_TILE)],
            out_hbm.at[c, pl.ds(s * ROWS_PER_TILE, ROWS_PER_TILE)],
        )

    return k


def _tc_matmul(x, w):
    def body(x_ref, w_ref, o_ref):
        o_ref[...] = jnp.dot(x_ref[...], w_ref[...],
                             preferred_element_type=jnp.float32)

    return pl.pallas_call(
        body,
        out_shape=jax.ShapeDtypeStruct((x.shape[0], w.shape[1]), jnp.float32),
    )(x, w)


def _tc_prescale(counts, x):
    def body(cnt_ref, x_ref, dinv_ref, xs_ref):
        deg = cnt_ref[0, :N, 0:1] + cnt_ref[1, :N, 0:1] + 1.0
        dinv = lax.rsqrt(deg)
        dinv_ref[...] = dinv
        xs_ref[...] = x_ref[...] * dinv

    return pl.pallas_call(
        body,
        out_shape=(
            jax.ShapeDtypeStruct((N, 1), jnp.float32),
            jax.ShapeDtypeStruct((N, x.shape[1]), jnp.float32),
        ),
    )(counts, x)


def _tc_layer1(acc, xs, dinv, w1, b1, g1, be1, w2):
    def body(acc_ref, xs_ref, dinv_ref, w1_ref, b_ref, g_ref, be_ref, w2_ref,
             o_ref):
        px = (acc_ref[0, :N] + acc_ref[1, :N] + xs_ref[...]) * dinv_ref[...]
        t = jnp.dot(px, w1_ref[...],
                    preferred_element_type=jnp.float32) + b_ref[...]
        mean = jnp.mean(t, axis=0, keepdims=True)
        var = jnp.mean((t - mean) ** 2, axis=0, keepdims=True)
        t = (t - mean) * lax.rsqrt(var + EPS) * g_ref[...] + be_ref[...]
        t = jnp.maximum(t, 0.0)
        o_ref[...] = jnp.dot(t, w2_ref[...],
                             preferred_element_type=jnp.float32) * dinv_ref[...]

    return pl.pallas_call(
        body,
        out_shape=jax.ShapeDtypeStruct((N, w2.shape[1]), jnp.float32),
    )(acc, xs, dinv, w1, b1.reshape(1, -1), g1.reshape(1, -1),
      be1.reshape(1, -1), w2)


def _tc_layer2(acc, hs2, dinv, b2, g2, be2):
    def body(acc_ref, hs_ref, dinv_ref, b_ref, g_ref, be_ref, o_ref):
        t = (acc_ref[0, :N] + acc_ref[1, :N] + hs_ref[...]) * dinv_ref[...] \
            + b_ref[...]
        mean = jnp.mean(t, axis=0, keepdims=True)
        var = jnp.mean((t - mean) ** 2, axis=0, keepdims=True)
        t = (t - mean) * lax.rsqrt(var + EPS) * g_ref[...] + be_ref[...]
        t = jnp.maximum(t, 0.0)
        o_ref[...] = t * dinv_ref[...]

    return pl.pallas_call(
        body,
        out_shape=jax.ShapeDtypeStruct((N, hs2.shape[1]), jnp.float32),
    )(acc, hs2, dinv, b2.reshape(1, -1), g2.reshape(1, -1), be2.reshape(1, -1))


def _tc_finish(acc, hs3, dinv, w3, b3, batch2d):
    def body(acc_ref, hs_ref, dinv_ref, w3_ref, b_ref, batch_ref, o_ref):
        ph = (acc_ref[0, :N] + acc_ref[1, :N] + hs_ref[...]) * dinv_ref[...]
        h3 = jnp.dot(ph, w3_ref[...],
                     preferred_element_type=jnp.float32) + b_ref[...]
        gids = lax.broadcasted_iota(jnp.int32, (N, G), 1)
        onehot = (batch_ref[...] == gids).astype(jnp.float32)
        sums = lax.dot_general(onehot, h3, (((0,), (0,)), ((), ())),
                               preferred_element_type=jnp.float32)
        cnts = jnp.sum(onehot, axis=0)[:, None]
        pooled = sums / jnp.maximum(cnts, 1.0)
        m = jnp.max(pooled, axis=1, keepdims=True)
        z = pooled - m
        o_ref[...] = z - jnp.log(jnp.sum(jnp.exp(z), axis=1, keepdims=True))

    return pl.pallas_call(
        body,
        out_shape=jax.ShapeDtypeStruct((G, w3.shape[1]), jnp.float32),
    )(acc, hs3, dinv, w3, b3.reshape(1, -1), batch2d)


def kernel(x, edge_index, batch, W1, b1, g1, be1, W2, b2, g2, be2, W3, b3):
    nt = NC * NS
    pad = E_PER_TILE - E // nt
    pr = jnp.broadcast_to(jnp.arange(pad, dtype=jnp.int32)[None], (nt, pad))
    src = jnp.concatenate([edge_index[0].reshape(nt, -1), pr], axis=1)
    dst = jnp.concatenate([edge_index[1].reshape(nt, -1), N + pr], axis=1)
    src = src.reshape(NC, NS, IG, GC, EB)
    dst = dst.reshape(NC, NS, IG, GC, EB)
    batch2d = batch.reshape(N, 1)

    counts = _sc_degree_kernel()(dst)
    dinv, xs = _tc_prescale(counts, x)

    acc1 = _sc_scatter_kernel(128)(xs, src, dst)
    hs2 = _tc_layer1(acc1, xs, dinv, W1, b1, g1, be1, W2)

    acc2 = _sc_scatter_kernel(128)(hs2, src, dst)
    hs3 = _tc_layer2(acc2, hs2, dinv, b2, g2, be2)

    acc3 = _sc_scatter_kernel(128)(hs3, src, dst)
    return _tc_finish(acc3, hs3, dinv, W3, b3, batch2d)


# speedup vs baseline: 3.6119x; 1.1353x over previous
import dataclasses
import functools

import jax
import jax.numpy as jnp
from jax import lax
from jax.experimental import pallas as pl
from jax.experimental.pallas import tpu as pltpu
from jax.experimental.pallas import tpu_sc as plsc

N = 10000
E = 320000
G = 128
EPS = 1e-5

NC = 2
NS = 16
EB = 64
E_PAD = 327680
E_PER_TILE = E_PAD // (NC * NS)
N_CHUNKS = E_PER_TILE // EB
NP = 10240
ROWS_PER_TILE = NP // NS
ZR = 16
IG = 4
GC = N_CHUNKS // IG
NB = 4

_MESH = plsc.VectorSubcoreMesh(core_axis_name="c", subcore_axis_name="s")

_CP = pltpu.CompilerParams()
if "needs_layout_passes" in pltpu.CompilerParams.__dataclass_fields__:
    _CP = dataclasses.replace(_CP, needs_layout_passes=False)


def _zero_fill(buf, rows, d):
    zero16 = jnp.zeros((16,), jnp.float32)

    @pl.loop(0, rows)
    def _(r):
        @pl.loop(0, d, step=16)
        def _(j):
            buf[r, pl.ds(j, 16)] = zero16


@functools.cache
def _sc_degree_kernel():
    @functools.partial(
        pl.kernel,
        mesh=_MESH,
        compiler_params=_CP,
        out_type=jax.ShapeDtypeStruct((NC, NS, NP), jnp.float32),
        scratch_types=[
            pltpu.VMEM((NP,), jnp.float32),
            pltpu.VMEM((GC * EB,), jnp.int32),
        ],
    )
    def k(dst_hbm, out_hbm, cnt_v, di_v):
        c = lax.axis_index("c")
        s = lax.axis_index("s")
        ones16 = jnp.ones((16,), jnp.float32)
        zero16 = jnp.zeros((16,), jnp.float32)

        @pl.loop(0, NP, step=16)
        def _(r):
            cnt_v[pl.ds(r, 16)] = zero16

        @pl.loop(0, IG)
        def _(g):
            pltpu.sync_copy(dst_hbm.at[c, s, g], di_v)

            @pl.loop(0, GC * EB, step=16)
            def _(j):
                dv = di_v[pl.ds(j, 16)]
                plsc.addupdate_scatter(cnt_v, [dv], ones16)

        pltpu.sync_copy(cnt_v, out_hbm.at[c, s])

    return k


@functools.cache
def _sc_scatter_kernel(d):
    @functools.partial(
        pl.kernel,
        mesh=_MESH,
        out_type=jax.ShapeDtypeStruct((NC, NP, d), jnp.float32),
        scratch_types=[
            pltpu.VMEM_SHARED((NP, d), jnp.float32),
        ] + [pltpu.VMEM((EB, d), jnp.float32) for _ in range(NB)] + [
            pltpu.VMEM((GC, EB), jnp.int32),
            pltpu.VMEM((GC, EB), jnp.int32),
            pltpu.VMEM((ZR, d), jnp.float32),
        ] + [pltpu.SemaphoreType.DMA for _ in range(NB)],
    )
    def k(hs_hbm, src_hbm, dst_hbm, out_hbm, acc_sh, b0, b1, b2, b3, si_v,
          di_v, z_v, s0, s1, s2, s3):
        c = lax.axis_index("c")
        s = lax.axis_index("s")
        bufs = [b0, b1, b2, b3]
        sems = [s0, s1, s2, s3]

        _zero_fill(z_v, ZR, d)

        @pl.loop(0, ROWS_PER_TILE, step=ZR)
        def _(r0):
            pltpu.sync_copy(z_v, acc_sh.at[pl.ds(s * ROWS_PER_TILE + r0, ZR)])

        plsc.subcore_barrier()

        @pl.loop(0, IG)
        def _(g):
            pltpu.sync_copy(src_hbm.at[c, s, g], si_v)
            pltpu.sync_copy(dst_hbm.at[c, s, g], di_v)
            for j in range(NB - 1):
                pltpu.async_copy(hs_hbm.at[si_v.at[j]], bufs[j], sems[j])

            @pl.loop(0, GC // NB)
            def _(kk):
                i0 = NB * kk
                for j in range(NB):
                    i = i0 + j
                    pltpu.make_async_copy(hs_hbm.at[si_v.at[i]], bufs[j],
                                          sems[j]).wait()
                    nxt = i + NB - 1
                    jn = (j + NB - 1) % NB

                    @pl.when(nxt < GC)
                    def _():
                        pltpu.async_copy(hs_hbm.at[si_v.at[nxt]], bufs[jn],
                                         sems[jn])

                    pltpu.sync_copy(bufs[j], acc_sh.at[di_v.at[i]], add=True)

        plsc.subcore_barrier()
        pltpu.sync_copy(
            acc_sh.at[pl.ds(s * ROWS_PER_TILE, ROWS_PER_TILE)],
            out_hbm.at[c, pl.ds(s * ROWS_PER_TILE, ROWS_PER_TILE)],
        )

    return k


def _tc_matmul(x, w):
    def body(x_ref, w_ref, o_ref):
        o_ref[...] = jnp.dot(x_ref[...], w_ref[...],
                             preferred_element_type=jnp.float32)

    return pl.pallas_call(
        body,
        out_shape=jax.ShapeDtypeStruct((x.shape[0], w.shape[1]), jnp.float32),
    )(x, w)


def _tc_prescale(deg_col, x):
    def body(cnt_ref, x_ref, dinv_ref, xs_ref):
        deg = cnt_ref[...] + 1.0
        dinv = lax.rsqrt(deg)
        dinv_ref[...] = dinv
        xs_ref[...] = x_ref[...] * dinv

    return pl.pallas_call(
        body,
        out_shape=(
            jax.ShapeDtypeStruct((N, 1), jnp.float32),
            jax.ShapeDtypeStruct((N, x.shape[1]), jnp.float32),
        ),
    )(deg_col, x)


def _tc_layer1(acc, xs, dinv, w1, b1, g1, be1, w2):
    def body(acc_ref, xs_ref, dinv_ref, w1_ref, b_ref, g_ref, be_ref, w2_ref,
             o_ref):
        px = (acc_ref[0, :N] + acc_ref[1, :N] + xs_ref[...]) * dinv_ref[...]
        t = jnp.dot(px, w1_ref[...],
                    preferred_element_type=jnp.float32) + b_ref[...]
        mean = jnp.mean(t, axis=0, keepdims=True)
        var = jnp.mean((t - mean) ** 2, axis=0, keepdims=True)
        t = (t - mean) * lax.rsqrt(var + EPS) * g_ref[...] + be_ref[...]
        t = jnp.maximum(t, 0.0)
        o_ref[...] = jnp.dot(t, w2_ref[...],
                             preferred_element_type=jnp.float32) * dinv_ref[...]

    return pl.pallas_call(
        body,
        out_shape=jax.ShapeDtypeStruct((N, w2.shape[1]), jnp.float32),
    )(acc, xs, dinv, w1, b1.reshape(1, -1), g1.reshape(1, -1),
      be1.reshape(1, -1), w2)


def _tc_layer2(acc, hs2, dinv, b2, g2, be2):
    def body(acc_ref, hs_ref, dinv_ref, b_ref, g_ref, be_ref, o_ref):
        t = (acc_ref[0, :N] + acc_ref[1, :N] + hs_ref[...]) * dinv_ref[...] \
            + b_ref[...]
        mean = jnp.mean(t, axis=0, keepdims=True)
        var = jnp.mean((t - mean) ** 2, axis=0, keepdims=True)
        t = (t - mean) * lax.rsqrt(var + EPS) * g_ref[...] + be_ref[...]
        t = jnp.maximum(t, 0.0)
        o_ref[...] = t * dinv_ref[...]

    return pl.pallas_call(
        body,
        out_shape=jax.ShapeDtypeStruct((N, hs2.shape[1]), jnp.float32),
    )(acc, hs2, dinv, b2.reshape(1, -1), g2.reshape(1, -1), be2.reshape(1, -1))


def _tc_finish(acc, hs3, dinv, w3, b3, batch2d):
    def body(acc_ref, hs_ref, dinv_ref, w3_ref, b_ref, batch_ref, o_ref):
        ph = (acc_ref[0, :N] + acc_ref[1, :N] + hs_ref[...]) * dinv_ref[...]
        h3 = jnp.dot(ph, w3_ref[...],
                     preferred_element_type=jnp.float32) + b_ref[...]
        gids = lax.broadcasted_iota(jnp.int32, (N, G), 1)
        onehot = (batch_ref[...] == gids).astype(jnp.float32)
        sums = lax.dot_general(onehot, h3, (((0,), (0,)), ((), ())),
                               preferred_element_type=jnp.float32)
        cnts = jnp.sum(onehot, axis=0)[:, None]
        pooled = sums / jnp.maximum(cnts, 1.0)
        m = jnp.max(pooled, axis=1, keepdims=True)
        z = pooled - m
        o_ref[...] = z - jnp.log(jnp.sum(jnp.exp(z), axis=1, keepdims=True))

    return pl.pallas_call(
        body,
        out_shape=jax.ShapeDtypeStruct((G, w3.shape[1]), jnp.float32),
    )(acc, hs3, dinv, w3, b3.reshape(1, -1), batch2d)


def kernel(x, edge_index, batch, W1, b1, g1, be1, W2, b2, g2, be2, W3, b3):
    nt = NC * NS
    pad = E_PER_TILE - E // nt
    pr = jnp.broadcast_to(jnp.arange(pad, dtype=jnp.int32)[None], (nt, pad))
    src = jnp.concatenate([edge_index[0].reshape(nt, -1), pr], axis=1)
    dst = jnp.concatenate([edge_index[1].reshape(nt, -1), N + pr], axis=1)
    src = src.reshape(NC, NS, IG, GC, EB)
    dst = dst.reshape(NC, NS, IG, GC, EB)
    batch2d = batch.reshape(N, 1)

    dst_deg = dst.reshape(NC, NS, IG, GC * EB)
    counts = _sc_degree_kernel()(dst_deg)
    deg_col = counts.sum(axis=(0, 1)).reshape(NP, 1)[:N]
    dinv, xs = _tc_prescale(deg_col, x)

    acc1 = _sc_scatter_kernel(128)(xs, src, dst)
    hs2 = _tc_layer1(acc1, xs, dinv, W1, b1, g1, be1, W2)

    acc2 = _sc_scatter_kernel(128)(hs2, src, dst)
    hs3 = _tc_layer2(acc2, hs2, dinv, b2, g2, be2)

    acc3 = _sc_scatter_kernel(128)(hs3, src, dst)
    return _tc_finish(acc3, hs3, dinv, W3, b3, batch2d)
